# baseline (device time: 13642 ns/iter reference)
import os

import jax
import jax.numpy as jnp
from jax import lax
from jax.experimental import pallas as pl
from jax.experimental.pallas import tpu as pltpu

_MODE = os.environ.get("KMODE", "full")

N_DEV = 4
B = 2
SQ = 128
SKV_PER = 128
HQ = 4
DH = 64
WINDOW = 128
D_MODEL = 512
D_QK = 256
ROWS_T = B * HQ * DH

SF = 0
SS = 1


def kernel(x, Wq, K_ext, V_ext, Wo):
    k16 = jnp.transpose(K_ext.astype(jnp.bfloat16),
                        (0, 2, 3, 1)).reshape(ROWS_T, SKV_PER)
    v16 = jnp.transpose(V_ext.astype(jnp.bfloat16),
                        (0, 2, 3, 1)).reshape(ROWS_T, SKV_PER)
    x = x.astype(jnp.bfloat16)
    Wq = Wq.astype(jnp.bfloat16)
    Wo = Wo.astype(jnp.bfloat16)

    def body(x_ref, wq_ref, k16_ref, v16_ref, wo_ref, out_ref, gath_ref,
             send_sems, recv_sems):
        my_pos = lax.axis_index("i")
        left = lax.rem(my_pos + N_DEV - 1, N_DEV)
        right = lax.rem(my_pos + 1, N_DEV)

        if _MODE != "nocomm":
            barrier_sem = pltpu.get_barrier_semaphore()
            pl.semaphore_signal(barrier_sem, inc=1, device_id=(left,),
                                device_id_type=pl.DeviceIdType.MESH)
            pl.semaphore_signal(barrier_sem, inc=1, device_id=(right,),
                                device_id_type=pl.DeviceIdType.MESH)
            pl.semaphore_wait(barrier_sem, 2)

        def rows(c):
            return slice(c * ROWS_T, (c + 1) * ROWS_T)

        def start_chunk(src_ref, dst_slot, c, send_idx, dev):
            pltpu.make_async_remote_copy(
                src_ref=src_ref,
                dst_ref=gath_ref.at[dst_slot, rows(c), :],
                send_sem=send_sems.at[send_idx],
                recv_sem=recv_sems.at[dst_slot, c],
                device_id=(dev,),
                device_id_type=pl.DeviceIdType.MESH,
            ).start()

        def wait_recv(slot, c):
            pltpu.make_async_remote_copy(
                src_ref=gath_ref.at[slot, rows(c), :],
                dst_ref=gath_ref.at[slot, rows(c), :],
                send_sem=send_sems.at[0],
                recv_sem=recv_sems.at[slot, c],
                device_id=(my_pos,), device_id_type=pl.DeviceIdType.MESH,
            ).wait_recv()

        def wait_send(send_idx):
            pltpu.make_async_remote_copy(
                src_ref=gath_ref.at[SF, rows(0), :],
                dst_ref=gath_ref.at[SF, rows(0), :],
                send_sem=send_sems.at[send_idx],
                recv_sem=recv_sems.at[SF, 0],
                device_id=(my_pos,), device_id_type=pl.DeviceIdType.MESH,
            ).wait_send()

        def stage_own():
            gath_ref[SF, rows(0), :] = k16_ref[...]
            gath_ref[SF, rows(1), :] = v16_ref[...]

        if _MODE == "nocomm":
            stage_own()
            gath_ref[SS] = gath_ref[SF]
        else:
            @pl.when(my_pos == 0)
            def _():
                start_chunk(k16_ref, SF, 0, 0, 3)
                start_chunk(v16_ref, SF, 1, 1, 3)
                start_chunk(k16_ref, SS, 0, 2, 1)
                start_chunk(v16_ref, SS, 1, 3, 1)
                stage_own()

            @pl.when(my_pos == 1)
            def _():
                start_chunk(k16_ref, SF, 0, 0, 2)
                start_chunk(v16_ref, SF, 1, 1, 2)
                start_chunk(k16_ref, SS, 0, 2, 0)
                start_chunk(v16_ref, SS, 1, 3, 0)
                stage_own()

        q_proj = [
            jnp.dot(x_ref[b], wq_ref[...],
                    preferred_element_type=jnp.float32
                    ).astype(jnp.bfloat16)
            for b in range(B)
        ]

        if _MODE != "nocomm":
            @pl.when(my_pos == 2)
            def _():
                wait_recv(SF, 0)
                start_chunk(gath_ref.at[SF, rows(0), :], SS, 0, 0, 3)
                wait_recv(SF, 1)
                start_chunk(gath_ref.at[SF, rows(1), :], SS, 1, 1, 3)

            @pl.when(my_pos == 3)
            def _():
                wait_recv(SF, 0)
                start_chunk(gath_ref.at[SF, rows(0), :], SS, 0, 0, 2)
                wait_recv(SF, 1)
                start_chunk(gath_ref.at[SF, rows(1), :], SS, 1, 1, 2)

        f_is_1 = (my_pos == 1) | (my_pos == 2)
        f_off = jnp.where(f_is_1, SKV_PER, 0)
        s_off = jnp.where(f_is_1, 0, SKV_PER)

        qi = lax.broadcasted_iota(jnp.int32, (SQ, SKV_PER), 0)
        kj = lax.broadcasted_iota(jnp.int32, (SQ, SKV_PER), 1)

        def block_k(slot, b, h):
            r0 = (b * HQ + h) * DH
            return gath_ref[slot, r0:r0 + DH, :]

        def block_v(slot, b, h):
            r0 = ROWS_T + (b * HQ + h) * DH
            return gath_ref[slot, r0:r0 + DH, :]

        def scores(slot, off, b, h):
            q_bh = q_proj[b][:, h * DH:(h + 1) * DH]
            s = jnp.dot(q_bh, block_k(slot, b, h),
                        preferred_element_type=jnp.float32) * 0.125
            return jnp.where(jnp.abs(qi - (kj + off)) <= WINDOW, s, -1e9)

        def pv(p, vT_blk):
            return lax.dot_general(
                p.astype(jnp.bfloat16), vT_blk, (((1,), (1,)), ((), ())),
                preferred_element_type=jnp.float32)

        if _MODE != "nocompute":
            parts = []
            for b in range(B):
                for h in range(HQ):
                    s_f = scores(SF, f_off, b, h)
                    m_f = jnp.max(s_f, axis=1, keepdims=True)
                    p_f = jnp.exp(s_f - m_f)
                    l_f = jnp.sum(p_f, axis=1, keepdims=True)
                    parts.append((m_f, l_f, pv(p_f, block_v(SF, b, h))))

        if _MODE != "nocomm":
            wait_recv(SS, 0)

        if _MODE != "nocompute":
            merged = []
            for b in range(B):
                for h in range(HQ):
                    m_f, l_f, c_f = parts[b * HQ + h]
                    s_s = scores(SS, s_off, b, h)
                    m = jnp.maximum(m_f,
                                    jnp.max(s_s, axis=1, keepdims=True))
                    alpha = jnp.exp(m_f - m)
                    p_s = jnp.exp(s_s - m)
                    l = l_f * alpha + jnp.sum(p_s, axis=1, keepdims=True)
                    merged.append((alpha, p_s, l, c_f))

        if _MODE != "nocomm":
            wait_recv(SS, 1)

        if _MODE == "nocompute":
            out_ref[...] = jnp.zeros((B, SQ, D_MODEL), jnp.float32)
        else:
            for b in range(B):
                ctx_heads = []
                for h in range(HQ):
                    alpha, p_s, l, c_f = merged[b * HQ + h]
                    c = c_f * alpha + pv(p_s, block_v(SS, b, h))
                    ctx_heads.append(c / l)
                ctx_b = jnp.concatenate(ctx_heads, axis=1)
                out_ref[b] = jnp.dot(ctx_b.astype(jnp.bfloat16),
                                     wo_ref[...],
                                     preferred_element_type=jnp.float32)

        if _MODE != "nocomm":
            @pl.when((my_pos == 0) | (my_pos == 1))
            def _():
                for i in range(4):
                    wait_send(i)

            @pl.when((my_pos == 2) | (my_pos == 3))
            def _():
                wait_send(0)
                wait_send(1)

    return pl.pallas_call(
        body,
        out_shape=jax.ShapeDtypeStruct((B, SQ, D_MODEL), jnp.float32),
        in_specs=[pl.BlockSpec(memory_space=pltpu.VMEM)] * 5,
        out_specs=pl.BlockSpec(memory_space=pltpu.VMEM),
        scratch_shapes=[
            pltpu.VMEM((2, 2 * ROWS_T, SKV_PER), jnp.bfloat16),
            pltpu.SemaphoreType.DMA((4,)),
            pltpu.SemaphoreType.DMA((2, 2)),
        ],
        compiler_params=pltpu.CompilerParams(collective_id=0),
    )(x, Wq, k16, v16, Wo)


# device time: 12583 ns/iter; 1.0842x vs baseline; 1.0842x over previous
import os

import jax
import jax.numpy as jnp
from jax import lax
from jax.experimental import pallas as pl
from jax.experimental.pallas import tpu as pltpu

_MODE = os.environ.get("KMODE", "full")

N_DEV = 4
B = 2
SQ = 128
SKV_PER = 128
HQ = 4
DH = 64
WINDOW = 128
D_MODEL = 512
D_QK = 256
ROWS_T = B * HQ * DH

SF = 0
SS = 1


def kernel(x, Wq, K_ext, V_ext, Wo):
    kT = jnp.transpose(K_ext, (0, 2, 3, 1)).reshape(ROWS_T, SKV_PER)
    vT = jnp.transpose(V_ext, (0, 2, 3, 1)).reshape(ROWS_T, SKV_PER)
    kv = jnp.concatenate([kT, vT], axis=0).astype(jnp.bfloat16)
    x = x.astype(jnp.bfloat16)
    Wq = Wq.astype(jnp.bfloat16)
    Wo = Wo.astype(jnp.bfloat16)

    def body(x_ref, wq_ref, kv_ref, wo_ref, out_ref, gath_ref,
             send_sems, recv_sems):
        my_pos = lax.axis_index("i")
        left = lax.rem(my_pos + N_DEV - 1, N_DEV)
        right = lax.rem(my_pos + 1, N_DEV)

        if _MODE != "nocomm":
            barrier_sem = pltpu.get_barrier_semaphore()
            pl.semaphore_signal(barrier_sem, inc=1, device_id=(left,),
                                device_id_type=pl.DeviceIdType.MESH)
            pl.semaphore_signal(barrier_sem, inc=1, device_id=(right,),
                                device_id_type=pl.DeviceIdType.MESH)
            pl.semaphore_wait(barrier_sem, 2)

        def rows(c):
            return slice(c * ROWS_T, (c + 1) * ROWS_T)

        def start_chunk(src_ref, dst_slot, c, send_idx, dev):
            pltpu.make_async_remote_copy(
                src_ref=src_ref,
                dst_ref=gath_ref.at[dst_slot, rows(c), :],
                send_sem=send_sems.at[send_idx],
                recv_sem=recv_sems.at[dst_slot, c],
                device_id=(dev,),
                device_id_type=pl.DeviceIdType.MESH,
            ).start()

        def wait_recv(slot, c):
            pltpu.make_async_remote_copy(
                src_ref=gath_ref.at[slot, rows(c), :],
                dst_ref=gath_ref.at[slot, rows(c), :],
                send_sem=send_sems.at[0],
                recv_sem=recv_sems.at[slot, c],
                device_id=(my_pos,), device_id_type=pl.DeviceIdType.MESH,
            ).wait_recv()

        def wait_send(send_idx):
            pltpu.make_async_remote_copy(
                src_ref=gath_ref.at[SF, rows(0), :],
                dst_ref=gath_ref.at[SF, rows(0), :],
                send_sem=send_sems.at[send_idx],
                recv_sem=recv_sems.at[SF, 0],
                device_id=(my_pos,), device_id_type=pl.DeviceIdType.MESH,
            ).wait_send()

        if _MODE == "nocomm":
            gath_ref[SF] = kv_ref[...]
            gath_ref[SS] = kv_ref[...]
        else:
            @pl.when(my_pos == 0)
            def _():
                start_chunk(kv_ref.at[rows(0), :], SF, 0, 0, 3)
                start_chunk(kv_ref.at[rows(1), :], SF, 1, 1, 3)
                start_chunk(kv_ref.at[rows(0), :], SS, 0, 2, 1)
                start_chunk(kv_ref.at[rows(1), :], SS, 1, 3, 1)
                gath_ref[SF] = kv_ref[...]

            @pl.when(my_pos == 1)
            def _():
                start_chunk(kv_ref.at[rows(0), :], SF, 0, 0, 2)
                start_chunk(kv_ref.at[rows(1), :], SF, 1, 1, 2)
                start_chunk(kv_ref.at[rows(0), :], SS, 0, 2, 0)
                start_chunk(kv_ref.at[rows(1), :], SS, 1, 3, 0)
                gath_ref[SF] = kv_ref[...]

        q_proj = [
            jnp.dot(x_ref[b], wq_ref[...],
                    preferred_element_type=jnp.float32
                    ).astype(jnp.bfloat16)
            for b in range(B)
        ]

        if _MODE != "nocomm":
            @pl.when(my_pos == 2)
            def _():
                wait_recv(SF, 0)
                start_chunk(gath_ref.at[SF, rows(0), :], SS, 0, 0, 3)
                wait_recv(SF, 1)
                start_chunk(gath_ref.at[SF, rows(1), :], SS, 1, 1, 3)

            @pl.when(my_pos == 3)
            def _():
                wait_recv(SF, 0)
                start_chunk(gath_ref.at[SF, rows(0), :], SS, 0, 0, 2)
                wait_recv(SF, 1)
                start_chunk(gath_ref.at[SF, rows(1), :], SS, 1, 1, 2)

        f_is_1 = (my_pos == 1) | (my_pos == 2)
        f_off = jnp.where(f_is_1, SKV_PER, 0)
        s_off = jnp.where(f_is_1, 0, SKV_PER)

        qi = lax.broadcasted_iota(jnp.int32, (SQ, SKV_PER), 0)
        kj = lax.broadcasted_iota(jnp.int32, (SQ, SKV_PER), 1)

        def block_k(slot, b, h):
            r0 = (b * HQ + h) * DH
            return gath_ref[slot, r0:r0 + DH, :]

        def block_v(slot, b, h):
            r0 = ROWS_T + (b * HQ + h) * DH
            return gath_ref[slot, r0:r0 + DH, :]

        def scores(slot, off, b, h):
            q_bh = q_proj[b][:, h * DH:(h + 1) * DH]
            s = jnp.dot(q_bh, block_k(slot, b, h),
                        preferred_element_type=jnp.float32) * 0.125
            return jnp.where(jnp.abs(qi - (kj + off)) <= WINDOW, s, -1e9)

        def pv(p, vT_blk):
            return lax.dot_general(
                p.astype(jnp.bfloat16), vT_blk, (((1,), (1,)), ((), ())),
                preferred_element_type=jnp.float32)

        if _MODE != "nocompute":
            parts = []
            for b in range(B):
                for h in range(HQ):
                    s_f = scores(SF, f_off, b, h)
                    m_f = jnp.max(s_f, axis=1, keepdims=True)
                    p_f = jnp.exp(s_f - m_f)
                    l_f = jnp.sum(p_f, axis=1, keepdims=True)
                    parts.append((m_f, l_f, pv(p_f, block_v(SF, b, h))))

        if _MODE != "nocomm":
            wait_recv(SS, 0)

        if _MODE != "nocompute":
            merged = []
            for b in range(B):
                for h in range(HQ):
                    m_f, l_f, c_f = parts[b * HQ + h]
                    s_s = scores(SS, s_off, b, h)
                    m = jnp.maximum(m_f,
                                    jnp.max(s_s, axis=1, keepdims=True))
                    alpha = jnp.exp(m_f - m)
                    p_s = jnp.exp(s_s - m)
                    l = l_f * alpha + jnp.sum(p_s, axis=1, keepdims=True)
                    merged.append((alpha, p_s, l, c_f))

        if _MODE != "nocomm":
            wait_recv(SS, 1)

        if _MODE == "nocompute":
            out_ref[...] = jnp.zeros((B, SQ, D_MODEL), jnp.float32)
        else:
            for b in range(B):
                ctx_heads = []
                for h in range(HQ):
                    alpha, p_s, l, c_f = merged[b * HQ + h]
                    c = c_f * alpha + pv(p_s, block_v(SS, b, h))
                    ctx_heads.append(c / l)
                ctx_b = jnp.concatenate(ctx_heads, axis=1)
                out_ref[b] = jnp.dot(ctx_b.astype(jnp.bfloat16),
                                     wo_ref[...],
                                     preferred_element_type=jnp.float32)

        if _MODE != "nocomm":
            @pl.when((my_pos == 0) | (my_pos == 1))
            def _():
                for i in range(4):
                    wait_send(i)

            @pl.when((my_pos == 2) | (my_pos == 3))
            def _():
                wait_send(0)
                wait_send(1)

    return pl.pallas_call(
        body,
        out_shape=jax.ShapeDtypeStruct((B, SQ, D_MODEL), jnp.float32),
        in_specs=[pl.BlockSpec(memory_space=pltpu.VMEM)] * 4,
        out_specs=pl.BlockSpec(memory_space=pltpu.VMEM),
        scratch_shapes=[
            pltpu.VMEM((2, 2 * ROWS_T, SKV_PER), jnp.bfloat16),
            pltpu.SemaphoreType.DMA((4,)),
            pltpu.SemaphoreType.DMA((2, 2)),
        ],
        compiler_params=pltpu.CompilerParams(collective_id=0),
    )(x, Wq, kv, Wo)
